# Initial kernel scaffold; baseline (speedup 1.0000x reference)
import jax
import jax.numpy as jnp
from jax.experimental import pallas as pl


def kernel(n_coords, ca_coords, c_coords, o_coords, params, sequence):
    raise NotImplementedError("write your pallas kernel here")


# speedup vs baseline: 1.6307x; 1.6307x over previous
import functools

import jax
import jax.numpy as jnp
import numpy as np
from jax.experimental import pallas as pl

_H = 128
_K = 48
_NRBF = 16
_RES = 1024
_RB = 128
_NB = _RES // _RB
_EB = _RB * _K
_E = _RES * _K
_ERAW = 256
_NTOK = 22
_NAA = 21
_BIGF = 3.0e38
_BIGI = 2 ** 30
_PREC = jax.lax.Precision.DEFAULT


def _dot(a, b):
    return jnp.dot(a, b, preferred_element_type=jnp.float32, precision=_PREC)


def _hilo_dot(oh, table):
    hi = table.astype(jnp.bfloat16)
    lo = (table - hi.astype(jnp.float32)).astype(jnp.bfloat16)
    ohb = oh.astype(jnp.bfloat16)
    return (jnp.dot(ohb, hi, preferred_element_type=jnp.float32)
            + jnp.dot(ohb, lo, preferred_element_type=jnp.float32))


def _gather_rows(idx_col, table_ref):
    m = idx_col.shape[0]
    lanes = jax.lax.broadcasted_iota(jnp.int32, (m, 128), 1)
    acc = None
    for c in range(_NB):
        oh = (idx_col == (lanes + c * 128)).astype(jnp.float32)
        part = _hilo_dot(oh, table_ref[c * 128:(c + 1) * 128, :])
        acc = part if acc is None else acc + part
    return acc


def _gather_local(rep_col, block):
    m = rep_col.shape[0]
    lanes = jax.lax.broadcasted_iota(jnp.int32, (m, 128), 1)
    oh = (rep_col == lanes).astype(jnp.float32)
    return _hilo_dot(oh, block)


def _ln(x, g, b):
    mu = jnp.mean(x, axis=-1, keepdims=True)
    var = jnp.mean((x - mu) ** 2, axis=-1, keepdims=True)
    return (x - mu) / jnp.sqrt(var + 1e-5) * g + b


def _knn_body(ca_pad_ref, ca_t_ref, out_ref):
    xi = ca_pad_ref[:, 0:1]
    yi = ca_pad_ref[:, 1:2]
    zi = ca_pad_ref[:, 2:3]
    dx = xi - ca_t_ref[0:1, :]
    dy = yi - ca_t_ref[1:2, :]
    dz = zi - ca_t_ref[2:3, :]
    d2 = dx * dx + dy * dy
    d2 = d2 + dz * dz
    lanes = jax.lax.broadcasted_iota(jnp.int32, (_RB, _RES), 1)
    work = d2
    cols = []
    for _ in range(_K):
        minv = jnp.min(work, axis=1, keepdims=True)
        sel = jnp.min(jnp.where(work == minv, lanes, _BIGI), axis=1,
                      keepdims=True)
        cols.append(sel)
        work = jnp.where(lanes == sel, _BIGF, work)
    out_ref[...] = jnp.concatenate(cols, axis=1)


def _ef_body(idx_ref, rep_ref, atoms_ref, atoms_blk_ref, mu_ref, out_ref):
    idx = idx_ref[...]
    rep = rep_ref[...]
    nbr = _gather_rows(idx, atoms_ref)
    slf = _gather_local(rep, atoms_blk_ref[...])
    mu = mu_ref[...]
    for a in range(4):
        for b in range(4):
            acc = None
            for c in range(3):
                dif = slf[:, a * 3 + c:a * 3 + c + 1] - \
                    nbr[:, b * 3 + c:b * 3 + c + 1]
                sq = dif * dif
                acc = sq if acc is None else acc + sq
            d = jnp.sqrt(acc + 1e-8)
            z = (d - mu) / 1.25
            p = a * 4 + b
            out_ref[:, p * _NRBF:(p + 1) * _NRBF] = jnp.exp(-(z * z))


def _premsg_body(h_ref, se_ref, wi_ref, b_ref, wj_ref, ws_ref,
                 ai_ref, aj_ref):
    h = h_ref[...]
    ai_ref[...] = _dot(h, wi_ref[...]) + b_ref[...]
    aj_ref[...] = _dot(h, wj_ref[...]) + _dot(se_ref[...], ws_ref[...])


def _edge_mlp(ai_blk_ref, aj_ref, idx_ref, rep_ref, e_ref,
              w1e_ref, w2_ref, b2_ref, w3_ref, b3_ref):
    idx = idx_ref[...]
    rep = rep_ref[...]
    gj = _gather_rows(idx, aj_ref)
    gi = _gather_local(rep, ai_blk_ref[...])
    t = gi + gj + _dot(e_ref[...], w1e_ref[...])
    t = jnp.maximum(t, 0.0)
    t = _dot(t, w2_ref[...]) + b2_ref[...]
    t = jnp.maximum(t, 0.0)
    return _dot(t, w3_ref[...]) + b3_ref[...]


def _msg_body(ai_blk_ref, aj_ref, idx_ref, rep_ref, e_ref,
              w1e_ref, w2_ref, b2_ref, w3_ref, b3_ref, out_ref):
    m = _edge_mlp(ai_blk_ref, aj_ref, idx_ref, rep_ref, e_ref,
                  w1e_ref, w2_ref, b2_ref, w3_ref, b3_ref)
    m3 = m.reshape(_RB, _K, _H)
    out_ref[...] = jnp.sum(m3, axis=1) * (1.0 / _K)


def _edgeup_body(ai_blk_ref, aj_ref, idx_ref, rep_ref, e_ref,
                 w1e_ref, w2_ref, b2_ref, w3_ref, b3_ref,
                 g_ref, bb_ref, out_ref):
    m = _edge_mlp(ai_blk_ref, aj_ref, idx_ref, rep_ref, e_ref,
                  w1e_ref, w2_ref, b2_ref, w3_ref, b3_ref)
    out_ref[...] = _ln(m, g_ref[...], bb_ref[...])


def _nodeup_body(h_ref, ms_ref, n1g_ref, n1b_ref, fw1_ref, fb1_ref,
                 fw2_ref, fb2_ref, n2g_ref, n2b_ref, out_ref):
    h = _ln(h_ref[...] + ms_ref[...], n1g_ref[...], n1b_ref[...])
    ff = jnp.maximum(_dot(h, fw1_ref[...]) + fb1_ref[...], 0.0)
    ff = _dot(ff, fw2_ref[...]) + fb2_ref[...]
    out_ref[...] = _ln(h + ff, n2g_ref[...], n2b_ref[...])


def _seqemb_body(seq_ref, tab_ref, out_ref):
    lanes = jax.lax.broadcasted_iota(jnp.int32, (_RES, 128), 1)
    oh = (seq_ref[...] == lanes).astype(jnp.float32)
    out_ref[...] = _hilo_dot(oh, tab_ref[...])


def _out_body(h_ref, w_ref, b_ref, out_ref):
    out_ref[...] = _dot(h_ref[...], w_ref[...]) + b_ref[...]


def _row(i):
    return (i, 0)


def _const(i):
    return (0, 0)


def _knn(ca_pad, ca_t):
    return pl.pallas_call(
        _knn_body,
        grid=(_NB,),
        in_specs=[pl.BlockSpec((_RB, 128), _row),
                  pl.BlockSpec((3, _RES), _const)],
        out_specs=pl.BlockSpec((_RB, _K), _row),
        out_shape=jax.ShapeDtypeStruct((_RES, _K), jnp.int32),
    )(ca_pad, ca_t)


def _edge_feat(idx_col, rep_col, atoms16, mu16):
    return pl.pallas_call(
        _ef_body,
        grid=(_NB,),
        in_specs=[pl.BlockSpec((_EB, 1), _row),
                  pl.BlockSpec((_EB, 1), _row),
                  pl.BlockSpec((_RES, 16), _const),
                  pl.BlockSpec((_RB, 16), _row),
                  pl.BlockSpec((1, 16), _const)],
        out_specs=pl.BlockSpec((_EB, _ERAW), _row),
        out_shape=jax.ShapeDtypeStruct((_E, _ERAW), jnp.float32),
    )(idx_col, rep_col, atoms16, atoms16, mu16)


def _premsg(h, se, wi, b1, wj, ws):
    return pl.pallas_call(
        _premsg_body,
        out_shape=[jax.ShapeDtypeStruct((_RES, _H), jnp.float32)] * 2,
    )(h, se, wi, b1.reshape(1, _H), wj, ws)


def _msg(ai, aj, idx_col, rep_col, e, w1e, w2, b2, w3, b3):
    ein = e.shape[-1]
    return pl.pallas_call(
        _msg_body,
        grid=(_NB,),
        in_specs=[pl.BlockSpec((_RB, _H), _row),
                  pl.BlockSpec((_RES, _H), _const),
                  pl.BlockSpec((_EB, 1), _row),
                  pl.BlockSpec((_EB, 1), _row),
                  pl.BlockSpec((_EB, ein), _row),
                  pl.BlockSpec((ein, _H), _const),
                  pl.BlockSpec((_H, _H), _const),
                  pl.BlockSpec((1, _H), _const),
                  pl.BlockSpec((_H, _H), _const),
                  pl.BlockSpec((1, _H), _const)],
        out_specs=pl.BlockSpec((_RB, _H), _row),
        out_shape=jax.ShapeDtypeStruct((_RES, _H), jnp.float32),
    )(ai, aj, idx_col, rep_col, e, w1e, w2, b2.reshape(1, _H), w3,
      b3.reshape(1, _H))


def _edgeup(ai, aj, idx_col, rep_col, e, w1e, w2, b2, w3, b3, g, bb):
    ein = e.shape[-1]
    return pl.pallas_call(
        _edgeup_body,
        grid=(_NB,),
        in_specs=[pl.BlockSpec((_RB, _H), _row),
                  pl.BlockSpec((_RES, _H), _const),
                  pl.BlockSpec((_EB, 1), _row),
                  pl.BlockSpec((_EB, 1), _row),
                  pl.BlockSpec((_EB, ein), _row),
                  pl.BlockSpec((ein, _H), _const),
                  pl.BlockSpec((_H, _H), _const),
                  pl.BlockSpec((1, _H), _const),
                  pl.BlockSpec((_H, _H), _const),
                  pl.BlockSpec((1, _H), _const),
                  pl.BlockSpec((1, _H), _const),
                  pl.BlockSpec((1, _H), _const)],
        out_specs=pl.BlockSpec((_EB, _H), _row),
        out_shape=jax.ShapeDtypeStruct((_E, _H), jnp.float32),
    )(ai, aj, idx_col, rep_col, e, w1e, w2, b2.reshape(1, _H), w3,
      b3.reshape(1, _H), g.reshape(1, _H), bb.reshape(1, _H))


def _nodeup(h, ms, n1g, n1b, fw1, fb1, fw2, fb2, n2g, n2b):
    return pl.pallas_call(
        _nodeup_body,
        out_shape=jax.ShapeDtypeStruct((_RES, _H), jnp.float32),
    )(h, ms, n1g.reshape(1, _H), n1b.reshape(1, _H), fw1,
      fb1.reshape(1, 4 * _H), fw2, fb2.reshape(1, _H), n2g.reshape(1, _H),
      n2b.reshape(1, _H))


def _seqemb(seq_col, tab_pad):
    return pl.pallas_call(
        _seqemb_body,
        out_shape=jax.ShapeDtypeStruct((_RES, _H), jnp.float32),
    )(seq_col, tab_pad)


def _outproj(h, w, b):
    return pl.pallas_call(
        _out_body,
        out_shape=jax.ShapeDtypeStruct((_RES, _NAA), jnp.float32),
    )(h, w, b.reshape(1, _NAA))


def kernel(n_coords, ca_coords, c_coords, o_coords, params, sequence):
    p = params
    ca_pad = jnp.pad(ca_coords, ((0, 0), (0, 125)))
    ca_t = ca_coords.T
    atoms16 = jnp.pad(
        jnp.concatenate([n_coords, ca_coords, c_coords, o_coords], axis=1),
        ((0, 0), (0, 4)))
    mu16 = jnp.linspace(2.0, 22.0, _NRBF,
                        dtype=jnp.float32).reshape(1, _NRBF)

    edge_idx = _knn(ca_pad, ca_t)
    idx_col = edge_idx.reshape(_E, 1)
    rep_col = (jnp.arange(_E, dtype=jnp.int32) // _K % _RB).reshape(_E, 1)

    edge_h = _edge_feat(idx_col, rep_col, atoms16, mu16)

    h = jnp.zeros((_RES, _H), jnp.float32)
    z128 = jnp.zeros((_RES, _H), jnp.float32)
    zw = jnp.zeros((_H, _H), jnp.float32)

    for i in range(3):
        pre = 'enc%d_' % i
        w1 = p[pre + 'mW1']
        ai, aj = _premsg(h, z128, w1[:_H], p[pre + 'mb1'], w1[_H:2 * _H], zw)
        ms = _msg(ai, aj, idx_col, rep_col, edge_h, w1[2 * _H:],
                  p[pre + 'mW2'], p[pre + 'mb2'], p[pre + 'mW3'],
                  p[pre + 'mb3'])
        h = _nodeup(h, ms, p[pre + 'n1g'], p[pre + 'n1b'], p[pre + 'fW1'],
                    p[pre + 'fb1'], p[pre + 'fW2'], p[pre + 'fb2'],
                    p[pre + 'n2g'], p[pre + 'n2b'])
        e1 = p[pre + 'eW1']
        ai, aj = _premsg(h, z128, e1[:_H], p[pre + 'eb1'], e1[_H:2 * _H], zw)
        edge_h = _edgeup(ai, aj, idx_col, rep_col, edge_h, e1[2 * _H:],
                         p[pre + 'eW2'], p[pre + 'eb2'], p[pre + 'eW3'],
                         p[pre + 'eb3'], p[pre + 'eng'], p[pre + 'enb'])

    tab_pad = jnp.pad(p['seq_emb'], ((0, 128 - _NTOK), (0, 0)))
    se = _seqemb(sequence.astype(jnp.int32).reshape(_RES, 1), tab_pad)

    for i in range(3):
        pre = 'dec%d_' % i
        w1 = p[pre + 'mW1']
        ai, aj = _premsg(h, se, w1[:_H], p[pre + 'mb1'], w1[_H:2 * _H],
                         w1[3 * _H:])
        ms = _msg(ai, aj, idx_col, rep_col, edge_h, w1[2 * _H:3 * _H],
                  p[pre + 'mW2'], p[pre + 'mb2'], p[pre + 'mW3'],
                  p[pre + 'mb3'])
        h = _nodeup(h, ms, p[pre + 'n1g'], p[pre + 'n1b'], p[pre + 'fW1'],
                    p[pre + 'fb1'], p[pre + 'fW2'], p[pre + 'fb2'],
                    p[pre + 'n2g'], p[pre + 'n2b'])

    return _outproj(h, p['out_W'], p['out_b'])


# speedup vs baseline: 2.2479x; 1.3785x over previous
import functools

import jax
import jax.numpy as jnp
import numpy as np
from jax import lax
from jax.experimental import pallas as pl
from jax.experimental.pallas import tpu as pltpu
from jax.experimental.pallas import tpu_sc as plsc

_H = 128
_K = 48
_NRBF = 16
_RES = 1024
_RB = 128
_NB = _RES // _RB
_EB = _RB * _K
_E = _RES * _K
_ERAW = 256
_NTOK = 22
_NAA = 21
_BIGF = 3.0e38
_BIGI = 2 ** 30
_PREC = jax.lax.Precision.DEFAULT


def _dot(a, b):
    return jnp.dot(a, b, preferred_element_type=jnp.float32, precision=_PREC)


def _hilo_dot(oh, table):
    hi = table.astype(jnp.bfloat16)
    lo = (table - hi.astype(jnp.float32)).astype(jnp.bfloat16)
    ohb = oh.astype(jnp.bfloat16)
    return (jnp.dot(ohb, hi, preferred_element_type=jnp.float32)
            + jnp.dot(ohb, lo, preferred_element_type=jnp.float32))


_SC_WORKERS = 32


@functools.cache
def _make_sc_gather(d, b, chunk):
    bpw = b // _SC_WORKERS
    nchunk = bpw // chunk
    mesh = plsc.VectorSubcoreMesh(core_axis_name="c", subcore_axis_name="s")

    @functools.partial(
        pl.kernel,
        out_type=jax.ShapeDtypeStruct((b, d), jnp.float32),
        mesh=mesh,
        scratch_types=[pltpu.VMEM((chunk,), jnp.int32),
                       pltpu.VMEM((chunk, d), jnp.float32),
                       pltpu.SemaphoreType.DMA],
    )
    def k(table_hbm, idx_hbm, out_hbm, idx_v, rows_v, sem):
        wid = lax.axis_index("s") * 2 + lax.axis_index("c")
        base = wid * bpw
        for c in range(nchunk):
            off = base + c * chunk
            pltpu.sync_copy(idx_hbm.at[pl.ds(off, chunk)], idx_v)
            pltpu.async_copy(table_hbm.at[idx_v], rows_v, sem).wait()
            pltpu.sync_copy(rows_v, out_hbm.at[pl.ds(off, chunk)])

    return k


def _sc_gather(table, idx, chunk):
    return _make_sc_gather(table.shape[1], idx.shape[0], chunk)(table, idx)


def _expand_i(blk):
    w = blk.shape[-1]
    return jnp.broadcast_to(blk[:, None, :], (_RB, _K, w)).reshape(_EB, w)


def _ln(x, g, b):
    mu = jnp.mean(x, axis=-1, keepdims=True)
    var = jnp.mean((x - mu) ** 2, axis=-1, keepdims=True)
    return (x - mu) / jnp.sqrt(var + 1e-5) * g + b


def _knn_body(ca_pad_ref, ca_t_ref, out_ref):
    xi = ca_pad_ref[:, 0:1]
    yi = ca_pad_ref[:, 1:2]
    zi = ca_pad_ref[:, 2:3]
    dx = xi - ca_t_ref[0:1, :]
    dy = yi - ca_t_ref[1:2, :]
    dz = zi - ca_t_ref[2:3, :]
    d2 = dx * dx + dy * dy
    d2 = d2 + dz * dz
    lanes = jax.lax.broadcasted_iota(jnp.int32, (_RB, _RES), 1)
    work = d2
    cols = []
    for _ in range(_K):
        minv = jnp.min(work, axis=1, keepdims=True)
        sel = jnp.min(jnp.where(work == minv, lanes, _BIGI), axis=1,
                      keepdims=True)
        cols.append(sel)
        work = jnp.where(lanes == sel, _BIGF, work)
    out_ref[...] = jnp.concatenate(cols, axis=1)


def _ef_body(nbr_ref, atoms_blk_ref, mu_ref, out_ref):
    nbr = nbr_ref[...]
    slf = _expand_i(atoms_blk_ref[...])
    mu = mu_ref[...]
    for a in range(4):
        for b in range(4):
            acc = None
            for c in range(3):
                dif = slf[:, a * 3 + c:a * 3 + c + 1] - \
                    nbr[:, b * 3 + c:b * 3 + c + 1]
                sq = dif * dif
                acc = sq if acc is None else acc + sq
            d = jnp.sqrt(acc + 1e-8)
            z = (d - mu) / 1.25
            p = a * 4 + b
            out_ref[:, p * _NRBF:(p + 1) * _NRBF] = jnp.exp(-(z * z))


def _premsg_body(h_ref, se_ref, wi_ref, b_ref, wj_ref, ws_ref,
                 ai_ref, aj_ref):
    h = h_ref[...]
    ai_ref[...] = _dot(h, wi_ref[...]) + b_ref[...]
    aj_ref[...] = _dot(h, wj_ref[...]) + _dot(se_ref[...], ws_ref[...])


def _edge_mlp(ai_blk_ref, gj_ref, e_ref,
              w1e_ref, w2_ref, b2_ref, w3_ref, b3_ref):
    gj = gj_ref[...]
    gi = _expand_i(ai_blk_ref[...])
    t = gi + gj + _dot(e_ref[...], w1e_ref[...])
    t = jnp.maximum(t, 0.0)
    t = _dot(t, w2_ref[...]) + b2_ref[...]
    t = jnp.maximum(t, 0.0)
    return _dot(t, w3_ref[...]) + b3_ref[...]


def _msg_body(ai_blk_ref, gj_ref, e_ref,
              w1e_ref, w2_ref, b2_ref, w3_ref, b3_ref, out_ref):
    m = _edge_mlp(ai_blk_ref, gj_ref, e_ref,
                  w1e_ref, w2_ref, b2_ref, w3_ref, b3_ref)
    m3 = m.reshape(_RB, _K, _H)
    out_ref[...] = jnp.sum(m3, axis=1) * (1.0 / _K)


def _edgeup_body(ai_blk_ref, gj_ref, e_ref,
                 w1e_ref, w2_ref, b2_ref, w3_ref, b3_ref,
                 g_ref, bb_ref, out_ref):
    m = _edge_mlp(ai_blk_ref, gj_ref, e_ref,
                  w1e_ref, w2_ref, b2_ref, w3_ref, b3_ref)
    out_ref[...] = _ln(m, g_ref[...], bb_ref[...])


def _nodeup_body(h_ref, ms_ref, n1g_ref, n1b_ref, fw1_ref, fb1_ref,
                 fw2_ref, fb2_ref, n2g_ref, n2b_ref, out_ref):
    h = _ln(h_ref[...] + ms_ref[...], n1g_ref[...], n1b_ref[...])
    ff = jnp.maximum(_dot(h, fw1_ref[...]) + fb1_ref[...], 0.0)
    ff = _dot(ff, fw2_ref[...]) + fb2_ref[...]
    out_ref[...] = _ln(h + ff, n2g_ref[...], n2b_ref[...])


def _out_body(h_ref, w_ref, b_ref, out_ref):
    out_ref[...] = _dot(h_ref[...], w_ref[...]) + b_ref[...]


def _row(i):
    return (i, 0)


def _const(i):
    return (0, 0)


def _knn(ca_pad, ca_t):
    return pl.pallas_call(
        _knn_body,
        grid=(_NB,),
        in_specs=[pl.BlockSpec((_RB, 128), _row),
                  pl.BlockSpec((3, _RES), _const)],
        out_specs=pl.BlockSpec((_RB, _K), _row),
        out_shape=jax.ShapeDtypeStruct((_RES, _K), jnp.int32),
    )(ca_pad, ca_t)


def _edge_feat(nbr, atoms16, mu16):
    return pl.pallas_call(
        _ef_body,
        grid=(_NB,),
        in_specs=[pl.BlockSpec((_EB, 16), _row),
                  pl.BlockSpec((_RB, 16), _row),
                  pl.BlockSpec((1, 16), _const)],
        out_specs=pl.BlockSpec((_EB, _ERAW), _row),
        out_shape=jax.ShapeDtypeStruct((_E, _ERAW), jnp.float32),
    )(nbr, atoms16, mu16)


def _premsg(h, se, wi, b1, wj, ws):
    return pl.pallas_call(
        _premsg_body,
        out_shape=[jax.ShapeDtypeStruct((_RES, _H), jnp.float32)] * 2,
    )(h, se, wi, b1.reshape(1, _H), wj, ws)


def _msg(ai, gj, e, w1e, w2, b2, w3, b3):
    ein = e.shape[-1]
    return pl.pallas_call(
        _msg_body,
        grid=(_NB,),
        in_specs=[pl.BlockSpec((_RB, _H), _row),
                  pl.BlockSpec((_EB, _H), _row),
                  pl.BlockSpec((_EB, ein), _row),
                  pl.BlockSpec((ein, _H), _const),
                  pl.BlockSpec((_H, _H), _const),
                  pl.BlockSpec((1, _H), _const),
                  pl.BlockSpec((_H, _H), _const),
                  pl.BlockSpec((1, _H), _const)],
        out_specs=pl.BlockSpec((_RB, _H), _row),
        out_shape=jax.ShapeDtypeStruct((_RES, _H), jnp.float32),
    )(ai, gj, e, w1e, w2, b2.reshape(1, _H), w3, b3.reshape(1, _H))


def _edgeup(ai, gj, e, w1e, w2, b2, w3, b3, g, bb):
    ein = e.shape[-1]
    return pl.pallas_call(
        _edgeup_body,
        grid=(_NB,),
        in_specs=[pl.BlockSpec((_RB, _H), _row),
                  pl.BlockSpec((_EB, _H), _row),
                  pl.BlockSpec((_EB, ein), _row),
                  pl.BlockSpec((ein, _H), _const),
                  pl.BlockSpec((_H, _H), _const),
                  pl.BlockSpec((1, _H), _const),
                  pl.BlockSpec((_H, _H), _const),
                  pl.BlockSpec((1, _H), _const),
                  pl.BlockSpec((1, _H), _const),
                  pl.BlockSpec((1, _H), _const)],
        out_specs=pl.BlockSpec((_EB, _H), _row),
        out_shape=jax.ShapeDtypeStruct((_E, _H), jnp.float32),
    )(ai, gj, e, w1e, w2, b2.reshape(1, _H), w3,
      b3.reshape(1, _H), g.reshape(1, _H), bb.reshape(1, _H))


def _nodeup(h, ms, n1g, n1b, fw1, fb1, fw2, fb2, n2g, n2b):
    return pl.pallas_call(
        _nodeup_body,
        out_shape=jax.ShapeDtypeStruct((_RES, _H), jnp.float32),
    )(h, ms, n1g.reshape(1, _H), n1b.reshape(1, _H), fw1,
      fb1.reshape(1, 4 * _H), fw2, fb2.reshape(1, _H), n2g.reshape(1, _H),
      n2b.reshape(1, _H))


def _outproj(h, w, b):
    return pl.pallas_call(
        _out_body,
        out_shape=jax.ShapeDtypeStruct((_RES, _NAA), jnp.float32),
    )(h, w, b.reshape(1, _NAA))


def kernel(n_coords, ca_coords, c_coords, o_coords, params, sequence):
    p = params
    ca_pad = jnp.pad(ca_coords, ((0, 0), (0, 125)))
    ca_t = ca_coords.T
    atoms16 = jnp.pad(
        jnp.concatenate([n_coords, ca_coords, c_coords, o_coords], axis=1),
        ((0, 0), (0, 4)))
    mu16 = jnp.linspace(2.0, 22.0, _NRBF,
                        dtype=jnp.float32).reshape(1, _NRBF)

    edge_idx = _knn(ca_pad, ca_t)
    idx = edge_idx.reshape(_E)

    atoms_pad = jnp.pad(atoms16, ((0, 0), (0, 112)))
    nbr = _sc_gather(atoms_pad, idx, 512)[:, :16]
    edge_h = _edge_feat(nbr, atoms16, mu16)

    h = jnp.zeros((_RES, _H), jnp.float32)
    z128 = jnp.zeros((_RES, _H), jnp.float32)
    zw = jnp.zeros((_H, _H), jnp.float32)

    for i in range(3):
        pre = 'enc%d_' % i
        w1 = p[pre + 'mW1']
        ai, aj = _premsg(h, z128, w1[:_H], p[pre + 'mb1'], w1[_H:2 * _H], zw)
        gj = _sc_gather(aj, idx, 512)
        ms = _msg(ai, gj, edge_h, w1[2 * _H:],
                  p[pre + 'mW2'], p[pre + 'mb2'], p[pre + 'mW3'],
                  p[pre + 'mb3'])
        h = _nodeup(h, ms, p[pre + 'n1g'], p[pre + 'n1b'], p[pre + 'fW1'],
                    p[pre + 'fb1'], p[pre + 'fW2'], p[pre + 'fb2'],
                    p[pre + 'n2g'], p[pre + 'n2b'])
        e1 = p[pre + 'eW1']
        ai, aj = _premsg(h, z128, e1[:_H], p[pre + 'eb1'], e1[_H:2 * _H], zw)
        gj = _sc_gather(aj, idx, 512)
        edge_h = _edgeup(ai, gj, edge_h, e1[2 * _H:],
                         p[pre + 'eW2'], p[pre + 'eb2'], p[pre + 'eW3'],
                         p[pre + 'eb3'], p[pre + 'eng'], p[pre + 'enb'])

    se = _sc_gather(p['seq_emb'], sequence.astype(jnp.int32), 32)

    for i in range(3):
        pre = 'dec%d_' % i
        w1 = p[pre + 'mW1']
        ai, aj = _premsg(h, se, w1[:_H], p[pre + 'mb1'], w1[_H:2 * _H],
                         w1[3 * _H:])
        gj = _sc_gather(aj, idx, 512)
        ms = _msg(ai, gj, edge_h, w1[2 * _H:3 * _H],
                  p[pre + 'mW2'], p[pre + 'mb2'], p[pre + 'mW3'],
                  p[pre + 'mb3'])
        h = _nodeup(h, ms, p[pre + 'n1g'], p[pre + 'n1b'], p[pre + 'fW1'],
                    p[pre + 'fb1'], p[pre + 'fW2'], p[pre + 'fb2'],
                    p[pre + 'n2g'], p[pre + 'n2b'])

    return _outproj(h, p['out_W'], p['out_b'])
